# baseline (speedup 1.0000x reference)
import jax
import jax.numpy as jnp
from jax import lax
from jax.experimental import pallas as pl
from jax.experimental.pallas import tpu as pltpu
from jax.experimental.pallas import tpu_sc as plsc

_D = 64
_V = 1_000_000
_B = 16384
_NW = 32
_WC = 400
_CW = _WC * _D
_NCH = _V // _WC
_BPW = _B // _NW


def _transpose_chunk(in_ref, ob_ref):
    lane = lax.iota(jnp.int32, 16)
    srcs = [(16 * j + lane) * _WC for j in range(4)]

    def col(v, _):
        for j in range(4):
            ob_ref[pl.ds(v * _D + 16 * j, 16)] = plsc.load_gather(
                in_ref, [srcs[j] + v]
            )
        return 0

    lax.fori_loop(0, _WC, col, 0)


def _transpose_body(w_hbm, wt_hbm, in0, in1, ob0, ob1, si0, si1, so0, so1):
    wid = lax.axis_index("s") * 2 + lax.axis_index("c")
    nt = 78 + jnp.where(wid < 4, 1, 0)

    def start_in(k, buf, sem):
        v0 = (wid + _NW * k) * _WC
        for d in range(_D):
            pltpu.make_async_copy(
                w_hbm.at[pl.ds(d * _V + v0, _WC)],
                buf.at[pl.ds(d * _WC, _WC)],
                sem,
            ).start()

    def drain_in(buf, sem):
        pltpu.make_async_copy(w_hbm.at[pl.ds(0, _CW)], buf, sem).wait()

    def out_copy(k, buf, sem):
        v0 = (wid + _NW * k) * _WC
        return pltpu.make_async_copy(
            buf, wt_hbm.at[pl.ds(v0 * _D, _CW)], sem
        )

    start_in(0, in0, si0)

    def step(m, _):
        k0 = 2 * m
        k1 = 2 * m + 1

        @pl.when(k0 < nt)
        def _():
            drain_in(in0, si0)

            @pl.when(k0 + 1 < nt)
            def _():
                start_in(k0 + 1, in1, si1)

            @pl.when(m > 0)
            def _():
                out_copy(0, ob0, so0).wait()

            _transpose_chunk(in0, ob0)
            out_copy(k0, ob0, so0).start()

        @pl.when(k1 < nt)
        def _():
            drain_in(in1, si1)

            @pl.when(k1 + 1 < nt)
            def _():
                start_in(k1 + 1, in0, si0)

            @pl.when(m > 0)
            def _():
                out_copy(0, ob1, so1).wait()

            _transpose_chunk(in1, ob1)
            out_copy(k1, ob1, so1).start()

        return 0

    lax.fori_loop(0, 40, step, 0)
    out_copy(0, ob0, so0).wait()
    out_copy(0, ob1, so1).wait()


def _gather_body(wt_hbm, idx_hbm, out_hbm, idx_v, p_v, h_v, rows_v, out_v, sem):
    wid = lax.axis_index("s") * 2 + lax.axis_index("c")
    base_b = wid * _BPW
    pltpu.sync_copy(idx_hbm.at[pl.ds(base_b, _BPW)], idx_v)

    def split(g, _):
        iv = idx_v[pl.ds(g * 16, 16)]
        p_v[pl.ds(g * 16, 16)] = iv >> 1
        h_v[pl.ds(g * 16, 16)] = iv & 1
        return 0

    lax.fori_loop(0, _BPW // 16, split, 0)

    pltpu.make_async_copy(wt_hbm.at[p_v], rows_v, sem).start()
    pltpu.make_async_copy(wt_hbm.at[p_v], rows_v, sem).wait()

    def extract(g, _):
        hvec = h_v[pl.ds(g * 16, 16)]
        for l in range(16):
            row = g * 16 + l
            off = hvec[l] * _D
            for j in range(4):
                out_v[pl.ds(row * _D + 16 * j, 16)] = rows_v[row, pl.ds(off + 16 * j, 16)]
        return 0

    lax.fori_loop(0, _BPW // 16, extract, 0)
    pltpu.sync_copy(out_v, out_hbm.at[pl.ds(base_b * _D, _BPW * _D)])


@jax.jit
def kernel(index, weights):
    idx32 = index.astype(jnp.int32)
    mesh = plsc.VectorSubcoreMesh(core_axis_name="c", subcore_axis_name="s")

    transpose = pl.kernel(
        _transpose_body,
        out_type=jax.ShapeDtypeStruct((_V * _D,), jnp.float32),
        mesh=mesh,
        compiler_params=pltpu.CompilerParams(needs_layout_passes=False),
        scratch_types=[
            pltpu.VMEM((_CW,), jnp.float32),
            pltpu.VMEM((_CW,), jnp.float32),
            pltpu.VMEM((_CW,), jnp.float32),
            pltpu.VMEM((_CW,), jnp.float32),
            pltpu.SemaphoreType.DMA,
            pltpu.SemaphoreType.DMA,
            pltpu.SemaphoreType.DMA,
            pltpu.SemaphoreType.DMA,
        ],
    )
    gather = pl.kernel(
        _gather_body,
        out_type=jax.ShapeDtypeStruct((_B * _D,), jnp.float32),
        mesh=mesh,
        compiler_params=pltpu.CompilerParams(needs_layout_passes=False),
        scratch_types=[
            pltpu.VMEM((_BPW,), jnp.int32),
            pltpu.VMEM((_BPW,), jnp.int32),
            pltpu.VMEM((_BPW,), jnp.int32),
            pltpu.VMEM((_BPW, 2 * _D), jnp.float32),
            pltpu.VMEM((_BPW * _D,), jnp.float32),
            pltpu.SemaphoreType.DMA,
        ],
    )

    wt = transpose(weights.reshape(_D * _V))
    out = gather(wt.reshape(_V // 2, 2 * _D), idx32)
    return out.reshape(_B, _D)


# speedup vs baseline: 1.1469x; 1.1469x over previous
import jax
import jax.numpy as jnp
from jax import lax
from jax.experimental import pallas as pl
from jax.experimental.pallas import tpu as pltpu
from jax.experimental.pallas import tpu_sc as plsc

_D = 64
_V = 1_000_000
_B = 16384
_NW = 32
_WC = 400
_CW = _WC * _D
_NCH = _V // _WC
_BPW = _B // _NW


def _transpose_chunk(in_ref, ob_ref):
    lane = lax.iota(jnp.int32, 16)
    srcs = [(16 * j + lane) * _WC for j in range(4)]

    def col(v, _):
        for j in range(4):
            ob_ref[pl.ds(v * _D + 16 * j, 16)] = plsc.load_gather(
                in_ref, [srcs[j] + v]
            )
        return 0

    lax.fori_loop(0, _WC, col, 0)


def _transpose_body(w_hbm, wt_hbm, in0, in1, ob0, ob1, si0, si1, so0, so1):
    wid = lax.axis_index("s") * 2 + lax.axis_index("c")
    nt = 78 + jnp.where(wid < 4, 1, 0)

    def start_in(k, buf, sem):
        v0 = (wid + _NW * k) * _WC
        for d in range(_D):
            pltpu.make_async_copy(
                w_hbm.at[pl.ds(d * _V + v0, _WC)],
                buf.at[pl.ds(d * _WC, _WC)],
                sem,
            ).start()

    def drain_in(buf, sem):
        pltpu.make_async_copy(w_hbm.at[pl.ds(0, _CW)], buf, sem).wait()

    def out_copy(k, buf, sem):
        v0 = (wid + _NW * k) * _WC
        return pltpu.make_async_copy(
            buf, wt_hbm.at[pl.ds(v0 * _D, _CW)], sem
        )

    start_in(0, in0, si0)

    def step(m, _):
        k0 = 2 * m
        k1 = 2 * m + 1

        @pl.when(k0 < nt)
        def _():
            drain_in(in0, si0)

            @pl.when(k0 + 1 < nt)
            def _():
                start_in(k0 + 1, in1, si1)

            @pl.when(m > 0)
            def _():
                out_copy(0, ob0, so0).wait()

            out_copy(k0, ob0, so0).start()

        @pl.when(k1 < nt)
        def _():
            drain_in(in1, si1)

            @pl.when(k1 + 1 < nt)
            def _():
                start_in(k1 + 1, in0, si0)

            @pl.when(m > 0)
            def _():
                out_copy(0, ob1, so1).wait()

            out_copy(k1, ob1, so1).start()

        return 0

    lax.fori_loop(0, 40, step, 0)
    out_copy(0, ob0, so0).wait()
    out_copy(0, ob1, so1).wait()


def _gather_body(wt_hbm, idx_hbm, out_hbm, idx_v, p_v, h_v, rows_v, out_v, sem):
    wid = lax.axis_index("s") * 2 + lax.axis_index("c")
    base_b = wid * _BPW
    pltpu.sync_copy(idx_hbm.at[pl.ds(base_b, _BPW)], idx_v)

    def split(g, _):
        iv = idx_v[pl.ds(g * 16, 16)]
        p_v[pl.ds(g * 16, 16)] = iv >> 1
        h_v[pl.ds(g * 16, 16)] = iv & 1
        return 0

    lax.fori_loop(0, _BPW // 16, split, 0)

    pltpu.make_async_copy(wt_hbm.at[p_v], rows_v, sem).start()
    pltpu.make_async_copy(wt_hbm.at[p_v], rows_v, sem).wait()

    def extract(g, _):
        hvec = h_v[pl.ds(g * 16, 16)]
        for l in range(16):
            row = g * 16 + l
            off = hvec[l] * _D
            for j in range(4):
                out_v[pl.ds(row * _D + 16 * j, 16)] = rows_v[row, pl.ds(off + 16 * j, 16)]
        return 0

    lax.fori_loop(0, _BPW // 16, extract, 0)
    pltpu.sync_copy(out_v, out_hbm.at[pl.ds(base_b * _D, _BPW * _D)])


@jax.jit
def kernel(index, weights):
    idx32 = index.astype(jnp.int32)
    mesh = plsc.VectorSubcoreMesh(core_axis_name="c", subcore_axis_name="s")

    transpose = pl.kernel(
        _transpose_body,
        out_type=jax.ShapeDtypeStruct((_V * _D,), jnp.float32),
        mesh=mesh,
        compiler_params=pltpu.CompilerParams(needs_layout_passes=False),
        scratch_types=[
            pltpu.VMEM((_CW,), jnp.float32),
            pltpu.VMEM((_CW,), jnp.float32),
            pltpu.VMEM((_CW,), jnp.float32),
            pltpu.VMEM((_CW,), jnp.float32),
            pltpu.SemaphoreType.DMA,
            pltpu.SemaphoreType.DMA,
            pltpu.SemaphoreType.DMA,
            pltpu.SemaphoreType.DMA,
        ],
    )
    gather = pl.kernel(
        _gather_body,
        out_type=jax.ShapeDtypeStruct((_B * _D,), jnp.float32),
        mesh=mesh,
        compiler_params=pltpu.CompilerParams(needs_layout_passes=False),
        scratch_types=[
            pltpu.VMEM((_BPW,), jnp.int32),
            pltpu.VMEM((_BPW,), jnp.int32),
            pltpu.VMEM((_BPW,), jnp.int32),
            pltpu.VMEM((_BPW, 2 * _D), jnp.float32),
            pltpu.VMEM((_BPW * _D,), jnp.float32),
            pltpu.SemaphoreType.DMA,
        ],
    )

    wt = transpose(weights.reshape(_D * _V))
    out = gather(wt.reshape(_V // 2, 2 * _D), idx32)
    return out.reshape(_B, _D)


# speedup vs baseline: 16.3482x; 14.2545x over previous
import jax
import jax.numpy as jnp
from jax import lax
from jax.experimental import pallas as pl
from jax.experimental.pallas import tpu as pltpu
from jax.experimental.pallas import tpu_sc as plsc

_D = 64
_V = 1_000_000
_B = 16384
_NW = 32
_BPW = _B // _NW
_HALF = _V // 2
_CB = 4096
_RB = _CB // 2
_GRID = (_V + _CB - 1) // _CB
_ROWS = _GRID * _RB


def _tc_transpose_body(x_ref, o_ref):
    eye = jnp.eye(_D, dtype=jnp.float32)

    def tr(x):
        return jax.lax.dot_general(
            x, eye, (((0,), (0,)), ((), ())),
            preferred_element_type=jnp.float32,
        )

    o_ref[:, 0:_D] = tr(x_ref[:, 0:_RB])
    o_ref[:, _D:2 * _D] = tr(x_ref[:, _RB:_CB])


def _gather_body(wt_hbm, idx_hbm, out_hbm, idx_v, p_v, h_v, rows_v, out_v, sem):
    wid = lax.axis_index("s") * 2 + lax.axis_index("c")
    base_b = wid * _BPW
    pltpu.sync_copy(idx_hbm.at[pl.ds(base_b, _BPW)], idx_v)

    def split(g, _):
        iv = idx_v[pl.ds(g * 16, 16)]
        c = iv >> 12
        j = iv & 4095
        p_v[pl.ds(g * 16, 16)] = (c << 11) | (j & 2047)
        h_v[pl.ds(g * 16, 16)] = (iv >> 11) & 1
        return 0

    lax.fori_loop(0, _BPW // 16, split, 0)

    pltpu.make_async_copy(wt_hbm.at[p_v], rows_v, sem).start()
    pltpu.make_async_copy(wt_hbm.at[p_v], rows_v, sem).wait()

    def extract(g, _):
        hvec = h_v[pl.ds(g * 16, 16)]
        for l in range(16):
            row = g * 16 + l
            off = hvec[l] * _D
            for j in range(4):
                out_v[pl.ds(row * _D + 16 * j, 16)] = rows_v[row, pl.ds(off + 16 * j, 16)]
        return 0

    lax.fori_loop(0, _BPW // 16, extract, 0)
    pltpu.sync_copy(out_v, out_hbm.at[pl.ds(base_b * _D, _BPW * _D)])


@jax.jit
def kernel(index, weights):
    idx32 = index.astype(jnp.int32)

    wt = pl.pallas_call(
        _tc_transpose_body,
        grid=(_GRID,),
        in_specs=[
            pl.BlockSpec((_D, _CB), lambda c: (0, c)),
        ],
        out_specs=pl.BlockSpec((_RB, 2 * _D), lambda c: (c, 0)),
        out_shape=jax.ShapeDtypeStruct((_ROWS, 2 * _D), jnp.float32),
    )(weights)

    gather = pl.kernel(
        _gather_body,
        out_type=jax.ShapeDtypeStruct((_B * _D,), jnp.float32),
        mesh=plsc.VectorSubcoreMesh(core_axis_name="c", subcore_axis_name="s"),
        compiler_params=pltpu.CompilerParams(needs_layout_passes=False),
        scratch_types=[
            pltpu.VMEM((_BPW,), jnp.int32),
            pltpu.VMEM((_BPW,), jnp.int32),
            pltpu.VMEM((_BPW,), jnp.int32),
            pltpu.VMEM((_BPW, 2 * _D), jnp.float32),
            pltpu.VMEM((_BPW * _D,), jnp.float32),
            pltpu.SemaphoreType.DMA,
        ],
    )

    out = gather(wt, idx32)
    return out.reshape(_B, _D)


# speedup vs baseline: 22.5571x; 1.3798x over previous
import jax
import jax.numpy as jnp
from jax import lax
from jax.experimental import pallas as pl
from jax.experimental.pallas import tpu as pltpu
from jax.experimental.pallas import tpu_sc as plsc

_D = 64
_V = 1_000_000
_B = 16384
_NW = 32
_BPW = _B // _NW
_HALF = _V // 2
_CB = 16384
_RB = _CB // 2
_GRID = (_V + _CB - 1) // _CB
_ROWS = _GRID * _RB


def _tc_transpose_body(x_ref, o_ref):
    eye = jnp.eye(_D, dtype=jnp.float32)

    def tr(x):
        return jax.lax.dot_general(
            x, eye, (((0,), (0,)), ((), ())),
            preferred_element_type=jnp.float32,
        )

    o_ref[:, 0:_D] = tr(x_ref[:, 0:_RB])
    o_ref[:, _D:2 * _D] = tr(x_ref[:, _RB:_CB])


def _gather_body(wt_hbm, idx_hbm, out_hbm, idx_v, p_v, h_v, rows_v, out_v, sem):
    wid = lax.axis_index("s") * 2 + lax.axis_index("c")
    base_b = wid * _BPW
    pltpu.sync_copy(idx_hbm.at[pl.ds(base_b, _BPW)], idx_v)

    def split(g, _):
        iv = idx_v[pl.ds(g * 16, 16)]
        c = iv >> 14
        j = iv & 16383
        p_v[pl.ds(g * 16, 16)] = (c << 13) | (j & 8191)
        h_v[pl.ds(g * 16, 16)] = (iv >> 13) & 1
        return 0

    lax.fori_loop(0, _BPW // 16, split, 0)

    pltpu.make_async_copy(wt_hbm.at[p_v], rows_v, sem).start()
    pltpu.make_async_copy(wt_hbm.at[p_v], rows_v, sem).wait()

    def extract(g, _):
        hvec = h_v[pl.ds(g * 16, 16)]
        for l in range(16):
            row = g * 16 + l
            off = hvec[l] * _D
            for j in range(4):
                out_v[pl.ds(row * _D + 16 * j, 16)] = rows_v[row, pl.ds(off + 16 * j, 16)]
        return 0

    lax.fori_loop(0, _BPW // 16, extract, 0)
    pltpu.sync_copy(out_v, out_hbm.at[pl.ds(base_b * _D, _BPW * _D)])


@jax.jit
def kernel(index, weights):
    idx32 = index.astype(jnp.int32)

    wt = pl.pallas_call(
        _tc_transpose_body,
        grid=(_GRID,),
        in_specs=[
            pl.BlockSpec((_D, _CB), lambda c: (0, c)),
        ],
        out_specs=pl.BlockSpec((_RB, 2 * _D), lambda c: (c, 0)),
        out_shape=jax.ShapeDtypeStruct((_ROWS, 2 * _D), jnp.float32),
    )(weights)

    gather = pl.kernel(
        _gather_body,
        out_type=jax.ShapeDtypeStruct((_B * _D,), jnp.float32),
        mesh=plsc.VectorSubcoreMesh(core_axis_name="c", subcore_axis_name="s"),
        compiler_params=pltpu.CompilerParams(needs_layout_passes=False),
        scratch_types=[
            pltpu.VMEM((_BPW,), jnp.int32),
            pltpu.VMEM((_BPW,), jnp.int32),
            pltpu.VMEM((_BPW,), jnp.int32),
            pltpu.VMEM((_BPW, 2 * _D), jnp.float32),
            pltpu.VMEM((_BPW * _D,), jnp.float32),
            pltpu.SemaphoreType.DMA,
        ],
    )

    out = gather(wt, idx32)
    return out.reshape(_B, _D)


# speedup vs baseline: 23.8270x; 1.0563x over previous
import jax
import jax.numpy as jnp
from jax import lax
from jax.experimental import pallas as pl
from jax.experimental.pallas import tpu as pltpu
from jax.experimental.pallas import tpu_sc as plsc

_D = 64
_V = 1_000_000
_B = 16384
_NW = 32
_BPW = _B // _NW
_HALF = _V // 2
_CB = 32768
_RB = _CB // 2
_GRID = (_V + _CB - 1) // _CB
_ROWS = _GRID * _RB


def _tc_transpose_body(x_ref, o_ref):
    eye = jnp.eye(_D, dtype=jnp.float32)

    def tr(x):
        return jax.lax.dot_general(
            x, eye, (((0,), (0,)), ((), ())),
            preferred_element_type=jnp.float32,
        )

    o_ref[:, 0:_D] = tr(x_ref[:, 0:_RB])
    o_ref[:, _D:2 * _D] = tr(x_ref[:, _RB:_CB])


def _gather_body(wt_hbm, idx_hbm, out_hbm, idx_v, p_v, h_v, rows_v, out_v, sem):
    wid = lax.axis_index("s") * 2 + lax.axis_index("c")
    base_b = wid * _BPW
    pltpu.sync_copy(idx_hbm.at[pl.ds(base_b, _BPW)], idx_v)

    def split(g, _):
        iv = idx_v[pl.ds(g * 16, 16)]
        c = iv >> 15
        j = iv & 32767
        p_v[pl.ds(g * 16, 16)] = (c << 14) | (j & 16383)
        h_v[pl.ds(g * 16, 16)] = (iv >> 14) & 1
        return 0

    lax.fori_loop(0, _BPW // 16, split, 0)

    pltpu.make_async_copy(wt_hbm.at[p_v], rows_v, sem).start()
    pltpu.make_async_copy(wt_hbm.at[p_v], rows_v, sem).wait()

    def extract(g, _):
        hvec = h_v[pl.ds(g * 16, 16)]
        for l in range(16):
            row = g * 16 + l
            off = hvec[l] * _D
            for j in range(4):
                out_v[pl.ds(row * _D + 16 * j, 16)] = rows_v[row, pl.ds(off + 16 * j, 16)]
        return 0

    lax.fori_loop(0, _BPW // 16, extract, 0)
    pltpu.sync_copy(out_v, out_hbm.at[pl.ds(base_b * _D, _BPW * _D)])


@jax.jit
def kernel(index, weights):
    idx32 = index.astype(jnp.int32)

    wt = pl.pallas_call(
        _tc_transpose_body,
        grid=(_GRID,),
        in_specs=[
            pl.BlockSpec((_D, _CB), lambda c: (0, c)),
        ],
        out_specs=pl.BlockSpec((_RB, 2 * _D), lambda c: (c, 0)),
        out_shape=jax.ShapeDtypeStruct((_ROWS, 2 * _D), jnp.float32),
    )(weights)

    gather = pl.kernel(
        _gather_body,
        out_type=jax.ShapeDtypeStruct((_B * _D,), jnp.float32),
        mesh=plsc.VectorSubcoreMesh(core_axis_name="c", subcore_axis_name="s"),
        compiler_params=pltpu.CompilerParams(needs_layout_passes=False),
        scratch_types=[
            pltpu.VMEM((_BPW,), jnp.int32),
            pltpu.VMEM((_BPW,), jnp.int32),
            pltpu.VMEM((_BPW,), jnp.int32),
            pltpu.VMEM((_BPW, 2 * _D), jnp.float32),
            pltpu.VMEM((_BPW * _D,), jnp.float32),
            pltpu.SemaphoreType.DMA,
        ],
    )

    out = gather(wt, idx32)
    return out.reshape(_B, _D)
